# baseline (speedup 1.0000x reference)
import functools

import jax
import jax.numpy as jnp
from jax import lax
from jax.experimental import pallas as pl
from jax.experimental.pallas import tpu as pltpu
from jax.experimental.pallas import tpu_sc as plsc

_NC, _NS = 2, 16
_NW = _NC * _NS
_K = 128
_NB = 8
_R = 1024


def _mesh():
    return plsc.VectorSubcoreMesh(core_axis_name="c", subcore_axis_name="s",
                                  num_cores=_NC, num_subcores=_NS)


def _sc_degree(dst2d, tpad):
    nw, nch, k = dst2d.shape
    rows = tpad // _NS
    nwave = nch // _NB

    @functools.partial(
        pl.kernel,
        out_type=(jax.ShapeDtypeStruct((tpad,), jnp.float32),
                  jax.ShapeDtypeStruct((tpad,), jnp.float32)),
        mesh=_mesh(),
        compiler_params=pltpu.CompilerParams(use_tc_tiling_on_sc=False),
        scratch_types=[
            pltpu.VMEM((nch, k), jnp.int32),
            pltpu.VMEM((k,), jnp.float32),
            pltpu.VMEM((k,), jnp.float32),
            pltpu.VMEM_SHARED((tpad,), jnp.float32),
            pltpu.SemaphoreType.DMA,
        ],
    )
    def deg_kernel(dst_hbm, out0_hbm, out1_hbm, idst2, ones_v, zv, acc_sh, ssem):
        c = lax.axis_index("c")
        s = lax.axis_index("s")
        w = s * _NC + c
        base = s * rows
        for i in range(k // 16):
            ones_v[pl.ds(i * 16, 16)] = jnp.full((16,), 1.0, jnp.float32)
            zv[pl.ds(i * 16, 16)] = jnp.zeros((16,), jnp.float32)
        for i in range(rows // k):
            pltpu.sync_copy(zv, acc_sh.at[pl.ds(base + i * k, k)])
        pltpu.sync_copy(dst_hbm.at[w], idst2)
        plsc.subcore_barrier()

        def wave(t, carry):
            j0 = t * _NB
            descs = [
                pltpu.async_copy(ones_v, acc_sh.at[idst2.at[j0 + b]], ssem, add=True)
                for b in range(_NB)
            ]
            for dsc in descs:
                dsc.wait()
            return carry

        lax.fori_loop(0, nwave, wave, 0)
        plsc.subcore_barrier()

        @pl.when(c == 0)
        def _():
            pltpu.sync_copy(acc_sh.at[pl.ds(base, rows)], out0_hbm.at[pl.ds(base, rows)])

        @pl.when(c == 1)
        def _():
            pltpu.sync_copy(acc_sh.at[pl.ds(base, rows)], out1_hbm.at[pl.ds(base, rows)])

    return deg_kernel(dst2d)


def _ring_scatter(p_hbm, isrc2, idst2, rows_v, acc_sh, gsems, ssems, nwave):
    for b in range(_NB):
        pltpu.async_copy(p_hbm.at[isrc2.at[b]], rows_v.at[b], gsems[b])

    def wave(t, carry):
        j0 = t * _NB
        sds = []
        for b in range(_NB):
            pltpu.make_async_copy(
                p_hbm.at[isrc2.at[j0 + b]], rows_v.at[b], gsems[b]).wait()
            sds.append(pltpu.async_copy(
                rows_v.at[b], acc_sh.at[idst2.at[j0 + b]], ssems[b], add=True))
        for b in range(_NB):
            sds[b].wait()

            @pl.when(t < nwave - 1)
            def _(b=b, j0=j0):
                pltpu.async_copy(
                    p_hbm.at[isrc2.at[j0 + _NB + b]], rows_v.at[b], gsems[b])
        return carry

    lax.fori_loop(0, nwave, wave, 0)


def _sc_scatter_split(src3d, dst3d, pa, pb):
    ns, nch, k = src3d.shape
    tpad, dh = pa.shape
    rows = tpad // _NS
    nwave = nch // _NB

    @functools.partial(
        pl.kernel,
        out_type=(jax.ShapeDtypeStruct((tpad, dh), jnp.float32),
                  jax.ShapeDtypeStruct((tpad, dh), jnp.float32)),
        mesh=_mesh(),
        compiler_params=pltpu.CompilerParams(use_tc_tiling_on_sc=False),
        scratch_types=[
            pltpu.VMEM((nch, k), jnp.int32),
            pltpu.VMEM((nch, k), jnp.int32),
            pltpu.VMEM((_NB, k, dh), jnp.float32),
            pltpu.VMEM((k, dh), jnp.float32),
            pltpu.VMEM_SHARED((tpad, dh), jnp.float32),
        ] + [pltpu.SemaphoreType.DMA] * (2 * _NB),
    )
    def scat_kernel(src_hbm, dst_hbm, pa_hbm, pb_hbm, out0_hbm, out1_hbm,
                    isrc2, idst2, rows_v, zt_v, acc_sh, *sems):
        gsems = sems[:_NB]
        ssems = sems[_NB:]
        c = lax.axis_index("c")
        s = lax.axis_index("s")
        base = s * rows
        for r in range(k):
            for j in range(dh // 16):
                zt_v[r, pl.ds(j * 16, 16)] = jnp.zeros((16,), jnp.float32)
        for i in range(rows // k):
            pltpu.sync_copy(zt_v, acc_sh.at[pl.ds(base + i * k, k)])
        pltpu.sync_copy(src_hbm.at[s], isrc2)
        pltpu.sync_copy(dst_hbm.at[s], idst2)
        plsc.subcore_barrier()

        @pl.when(c == 0)
        def _():
            _ring_scatter(pa_hbm, isrc2, idst2, rows_v, acc_sh, gsems, ssems, nwave)

        @pl.when(c == 1)
        def _():
            _ring_scatter(pb_hbm, isrc2, idst2, rows_v, acc_sh, gsems, ssems, nwave)

        plsc.subcore_barrier()

        @pl.when(c == 0)
        def _():
            pltpu.sync_copy(acc_sh.at[pl.ds(base, rows)], out0_hbm.at[pl.ds(base, rows)])

        @pl.when(c == 1)
        def _():
            pltpu.sync_copy(acc_sh.at[pl.ds(base, rows)], out1_hbm.at[pl.ds(base, rows)])

    return scat_kernel(src3d, dst3d, pa, pb)


def _sc_scatter(src2d, dst2d, p):
    nw, nch, k = src2d.shape
    tpad, d = p.shape
    rows = tpad // _NS
    nwave = nch // _NB

    @functools.partial(
        pl.kernel,
        out_type=(jax.ShapeDtypeStruct((tpad, d), jnp.float32),
                  jax.ShapeDtypeStruct((tpad, d), jnp.float32)),
        mesh=_mesh(),
        compiler_params=pltpu.CompilerParams(use_tc_tiling_on_sc=False),
        scratch_types=[
            pltpu.VMEM((nch, k), jnp.int32),
            pltpu.VMEM((nch, k), jnp.int32),
            pltpu.VMEM((_NB, k, d), jnp.float32),
            pltpu.VMEM((k, d), jnp.float32),
            pltpu.VMEM_SHARED((tpad, d), jnp.float32),
        ] + [pltpu.SemaphoreType.DMA] * (2 * _NB),
    )
    def scat_kernel(src_hbm, dst_hbm, p_hbm, out0_hbm, out1_hbm,
                    isrc2, idst2, rows_v, zt_v, acc_sh, *sems):
        gsems = sems[:_NB]
        ssems = sems[_NB:]
        c = lax.axis_index("c")
        s = lax.axis_index("s")
        w = s * _NC + c
        base = s * rows
        for r in range(k):
            for j in range(d // 16):
                zt_v[r, pl.ds(j * 16, 16)] = jnp.zeros((16,), jnp.float32)
        for i in range(rows // k):
            pltpu.sync_copy(zt_v, acc_sh.at[pl.ds(base + i * k, k)])
        pltpu.sync_copy(src_hbm.at[w], isrc2)
        pltpu.sync_copy(dst_hbm.at[w], idst2)
        plsc.subcore_barrier()
        _ring_scatter(p_hbm, isrc2, idst2, rows_v, acc_sh, gsems, ssems, nwave)
        plsc.subcore_barrier()

        @pl.when(c == 0)
        def _():
            pltpu.sync_copy(acc_sh.at[pl.ds(base, rows)], out0_hbm.at[pl.ds(base, rows)])

        @pl.when(c == 1)
        def _():
            pltpu.sync_copy(acc_sh.at[pl.ds(base, rows)], out1_hbm.at[pl.ds(base, rows)])

    return scat_kernel(src2d, dst2d, p)


def _tc_matmul(x, w1):
    n, d_in = x.shape
    d_out = w1.shape[1]

    def body(x_ref, w_ref, h_ref):
        h_ref[...] = jnp.dot(x_ref[...], w_ref[...], preferred_element_type=jnp.float32)

    return pl.pallas_call(
        body,
        grid=(n // _R,),
        in_specs=[
            pl.BlockSpec((_R, d_in), lambda i: (i, 0)),
            pl.BlockSpec((d_in, d_out), lambda i: (0, 0)),
        ],
        out_specs=pl.BlockSpec((_R, d_out), lambda i: (i, 0)),
        out_shape=jax.ShapeDtypeStruct((n, d_out), jnp.float32),
    )(x, w1)


def _tc_scale(h1, dg0, dg1):
    n, d_out = h1.shape
    dh = d_out // 2

    def body(h_ref, d0_ref, d1_ref, pa_ref, pb_ref, dis_ref):
        deg = d0_ref[...] + d1_ref[...] + 1.0
        dis = lax.rsqrt(deg)
        p = h_ref[...] * dis
        pa_ref[...] = p[:, :dh]
        pb_ref[...] = p[:, dh:]
        dis_ref[...] = dis

    return pl.pallas_call(
        body,
        grid=(n // _R,),
        in_specs=[
            pl.BlockSpec((_R, d_out), lambda i: (i, 0)),
            pl.BlockSpec((_R, 1), lambda i: (i, 0)),
            pl.BlockSpec((_R, 1), lambda i: (i, 0)),
        ],
        out_specs=[
            pl.BlockSpec((_R, dh), lambda i: (i, 0)),
            pl.BlockSpec((_R, dh), lambda i: (i, 0)),
            pl.BlockSpec((_R, 1), lambda i: (i, 0)),
        ],
        out_shape=[
            jax.ShapeDtypeStruct((n, dh), jnp.float32),
            jax.ShapeDtypeStruct((n, dh), jnp.float32),
            jax.ShapeDtypeStruct((n, 1), jnp.float32),
        ],
    )(h1, dg0, dg1)


def _tc_mid(pa, pb, sa, sb, dis, ba, bb, wa, wb, split_out, out_width):
    n, dh = pa.shape
    d2 = wa.shape[1]

    def body(pa_ref, pb_ref, sa_ref, sb_ref, dis_ref, ba_ref, bb_ref,
             wa_ref, wb_ref, *out_refs):
        dis = dis_ref[...]
        ha = jnp.maximum((sa_ref[...] + pa_ref[...]) * dis + ba_ref[...], 0.0)
        hb = jnp.maximum((sb_ref[...] + pb_ref[...]) * dis + bb_ref[...], 0.0)
        q = (jnp.dot(ha, wa_ref[...], preferred_element_type=jnp.float32)
             + jnp.dot(hb, wb_ref[...], preferred_element_type=jnp.float32)) * dis
        if split_out:
            out_refs[0][...] = q[:, :d2 // 2]
            out_refs[1][...] = q[:, d2 // 2:]
        else:
            out_refs[0][...] = jnp.broadcast_to(q, (q.shape[0], out_width))

    if split_out:
        out_specs = [pl.BlockSpec((_R, d2 // 2), lambda i: (i, 0)),
                     pl.BlockSpec((_R, d2 // 2), lambda i: (i, 0))]
        out_shape = [jax.ShapeDtypeStruct((n, d2 // 2), jnp.float32),
                     jax.ShapeDtypeStruct((n, d2 // 2), jnp.float32)]
    else:
        out_specs = pl.BlockSpec((_R, out_width), lambda i: (i, 0))
        out_shape = jax.ShapeDtypeStruct((n, out_width), jnp.float32)

    return pl.pallas_call(
        body,
        grid=(n // _R,),
        in_specs=[
            pl.BlockSpec((_R, dh), lambda i: (i, 0)),
            pl.BlockSpec((_R, dh), lambda i: (i, 0)),
            pl.BlockSpec((_R, dh), lambda i: (i, 0)),
            pl.BlockSpec((_R, dh), lambda i: (i, 0)),
            pl.BlockSpec((_R, 1), lambda i: (i, 0)),
            pl.BlockSpec((1, dh), lambda i: (0, 0)),
            pl.BlockSpec((1, dh), lambda i: (0, 0)),
            pl.BlockSpec((dh, d2), lambda i: (0, 0)),
            pl.BlockSpec((dh, d2), lambda i: (0, 0)),
        ],
        out_specs=out_specs,
        out_shape=out_shape,
    )(pa, pb, sa, sb, dis, ba, bb, wa, wb)


def _tc_last(p16, a0, a1, dis, b3):
    n, d = p16.shape

    def body(p_ref, a0_ref, a1_ref, dis_ref, b_ref, o_ref):
        acc = a0_ref[...][:, :1] + a1_ref[...][:, :1] + p_ref[...][:, :1]
        v = acc * dis_ref[...] + b_ref[...]
        o_ref[...] = jax.nn.sigmoid(v)

    return pl.pallas_call(
        body,
        grid=(n // _R,),
        in_specs=[
            pl.BlockSpec((_R, d), lambda i: (i, 0)),
            pl.BlockSpec((_R, d), lambda i: (i, 0)),
            pl.BlockSpec((_R, d), lambda i: (i, 0)),
            pl.BlockSpec((_R, 1), lambda i: (i, 0)),
            pl.BlockSpec((1, 1), lambda i: (0, 0)),
        ],
        out_specs=pl.BlockSpec((_R, 1), lambda i: (i, 0)),
        out_shape=jax.ShapeDtypeStruct((n, 1), jnp.float32),
    )(p16, a0, a1, dis, b3)


def kernel(x, edge_index, W1, b1, W2, b2, W3, b3):
    n, d_feat = x.shape
    e = edge_index.shape[1]
    tpad = ((n + _R - 1) // _R) * _R
    assert tpad % (_NS * 8) == 0

    ew = e // _NW
    ew_pad = ((ew + _K * _NB - 1) // (_K * _NB)) * (_K * _NB)
    epad = ew_pad * _NW
    nch_w = ew_pad // _K
    nch_s = (epad // _NS) // _K
    n_dummy = ew_pad - ew
    assert e % _NW == 0 and n_dummy <= tpad - n and nch_s % _NB == 0
    pad_rows = n + jnp.arange(n_dummy, dtype=jnp.int32) % (tpad - n)

    def _prep(idx):
        w2 = idx.astype(jnp.int32).reshape(_NW, ew)
        padb = jnp.broadcast_to(pad_rows, (_NW, n_dummy))
        flat = jnp.concatenate([w2, padb], axis=1)
        return (flat.reshape(_NW, nch_w, _K), flat.reshape(_NS, nch_s, _K))

    src2d, src3d = _prep(edge_index[0])
    dst2d, dst3d = _prep(edge_index[1])
    x_pad = jnp.zeros((tpad, d_feat), jnp.float32).at[:n].set(x)

    d1h = W1.shape[1] // 2
    d2h = W2.shape[1] // 2

    degp0, degp1 = _sc_degree(dst2d, tpad)
    dg0 = degp0.reshape(tpad, 1)
    dg1 = degp1.reshape(tpad, 1)
    h1 = _tc_matmul(x_pad, W1)
    p1a, p1b, dis = _tc_scale(h1, dg0, dg1)
    s1a, s1b = _sc_scatter_split(src3d, dst3d, p1a, p1b)
    p2a, p2b = _tc_mid(p1a, p1b, s1a, s1b, dis,
                       b1.reshape(1, -1)[:, :d1h], b1.reshape(1, -1)[:, d1h:],
                       W2[:d1h], W2[d1h:], True, 0)
    s2a, s2b = _sc_scatter_split(src3d, dst3d, p2a, p2b)
    p3 = _tc_mid(p2a, p2b, s2a, s2b, dis,
                 b2.reshape(1, -1)[:, :d2h], b2.reshape(1, -1)[:, d2h:],
                 W3[:d2h], W3[d2h:], False, 16)
    a0, a1 = _sc_scatter(src2d, dst2d, p3)
    out = _tc_last(p3, a0, a1, dis, b3.reshape(1, 1))
    return out[:n]


# speedup vs baseline: 1.0326x; 1.0326x over previous
import functools

import jax
import jax.numpy as jnp
from jax import lax
from jax.experimental import pallas as pl
from jax.experimental.pallas import tpu as pltpu
from jax.experimental.pallas import tpu_sc as plsc

_NC, _NS = 2, 16
_NW = _NC * _NS
_K = 128
_G = 1
_NB = 8
_R = 1024


def _mesh():
    return plsc.VectorSubcoreMesh(core_axis_name="c", subcore_axis_name="s",
                                  num_cores=_NC, num_subcores=_NS)


def _sc_degree(dst3d, tpad):
    ns, nch_s, k = dst3d.shape
    nch = nch_s // _NC
    rows = tpad // _NS
    nwave = nch // (_G * _NB)

    @functools.partial(
        pl.kernel,
        out_type=(jax.ShapeDtypeStruct((tpad,), jnp.float32),
                  jax.ShapeDtypeStruct((tpad,), jnp.float32)),
        mesh=_mesh(),
        compiler_params=pltpu.CompilerParams(use_tc_tiling_on_sc=False),
        scratch_types=[
            pltpu.VMEM((nch, k), jnp.int32),
            pltpu.VMEM((k,), jnp.float32),
            pltpu.VMEM((k,), jnp.float32),
            pltpu.VMEM_SHARED((tpad,), jnp.float32),
            pltpu.SemaphoreType.DMA,
        ],
    )
    def deg_kernel(dst_hbm, out0_hbm, out1_hbm, idst2, ones_v, zv, acc_sh, ssem):
        c = lax.axis_index("c")
        s = lax.axis_index("s")
        w = s * _NC + c
        base = s * rows
        for i in range(k // 16):
            ones_v[pl.ds(i * 16, 16)] = jnp.full((16,), 1.0, jnp.float32)
            zv[pl.ds(i * 16, 16)] = jnp.zeros((16,), jnp.float32)
        for i in range(rows // k):
            pltpu.sync_copy(zv, acc_sh.at[pl.ds(base + i * k, k)])
        pltpu.sync_copy(dst_hbm.at[w // _NC, pl.ds((w % _NC) * nch, nch)], idst2)
        plsc.subcore_barrier()

        def wave(t, carry):
            g0 = t * _NB
            descs = [
                pltpu.async_copy(ones_v, acc_sh.at[idst2.at[g0 + b]], ssem, add=True)
                for b in range(_NB)
            ]
            for dsc in descs:
                dsc.wait()
            return carry

        lax.fori_loop(0, nwave, wave, 0)
        plsc.subcore_barrier()

        @pl.when(c == 0)
        def _():
            pltpu.sync_copy(acc_sh.at[pl.ds(base, rows)], out0_hbm.at[pl.ds(base, rows)])

        @pl.when(c == 1)
        def _():
            pltpu.sync_copy(acc_sh.at[pl.ds(base, rows)], out1_hbm.at[pl.ds(base, rows)])

    return deg_kernel(dst3d)


def _ring_scatter(p_hbm, isrc2, idst2, rows_v, acc_sh, gsems, ssems, nwave):
    for b in range(_NB):
        pltpu.async_copy(p_hbm.at[isrc2.at[b]], rows_v.at[b], gsems[b])

    def wave(t, carry):
        g0 = t * _NB
        sds = []
        for b in range(_NB):
            pltpu.make_async_copy(
                p_hbm.at[isrc2.at[g0 + b]], rows_v.at[b], gsems[b]).wait()
            sds.append(pltpu.async_copy(
                rows_v.at[b], acc_sh.at[idst2.at[g0 + b]], ssems[b], add=True))
        for b in range(_NB):
            sds[b].wait()

            @pl.when(t < nwave - 1)
            def _(b=b, g0=g0):
                pltpu.async_copy(
                    p_hbm.at[isrc2.at[g0 + _NB + b]], rows_v.at[b], gsems[b])
        return carry

    lax.fori_loop(0, nwave, wave, 0)


def _sc_scatter_split(src3d, dst3d, pa, pb):
    ns, nch, k = src3d.shape
    tpad, dh = pa.shape
    rows = tpad // _NS
    nwave = nch // (_G * _NB)

    @functools.partial(
        pl.kernel,
        out_type=(jax.ShapeDtypeStruct((tpad, dh), jnp.float32),
                  jax.ShapeDtypeStruct((tpad, dh), jnp.float32)),
        mesh=_mesh(),
        compiler_params=pltpu.CompilerParams(use_tc_tiling_on_sc=False),
        scratch_types=[
            pltpu.VMEM((nch, k), jnp.int32),
            pltpu.VMEM((nch, k), jnp.int32),
            pltpu.VMEM((_NB, k, dh), jnp.float32),
            pltpu.VMEM((k, dh), jnp.float32),
            pltpu.VMEM_SHARED((tpad, dh), jnp.float32),
        ] + [pltpu.SemaphoreType.DMA] * (2 * _NB),
    )
    def scat_kernel(src_hbm, dst_hbm, pa_hbm, pb_hbm, out0_hbm, out1_hbm,
                    isrc2, idst2, rows_v, zt_v, acc_sh, *sems):
        gsems = sems[:_NB]
        ssems = sems[_NB:]
        c = lax.axis_index("c")
        s = lax.axis_index("s")
        base = s * rows
        for r in range(k):
            for j in range(dh // 16):
                zt_v[r, pl.ds(j * 16, 16)] = jnp.zeros((16,), jnp.float32)
        for i in range(rows // k):
            pltpu.sync_copy(zt_v, acc_sh.at[pl.ds(base + i * k, k)])
        pltpu.sync_copy(src_hbm.at[s], isrc2)
        pltpu.sync_copy(dst_hbm.at[s], idst2)
        plsc.subcore_barrier()

        @pl.when(c == 0)
        def _():
            _ring_scatter(pa_hbm, isrc2, idst2, rows_v, acc_sh, gsems, ssems, nwave)

        @pl.when(c == 1)
        def _():
            _ring_scatter(pb_hbm, isrc2, idst2, rows_v, acc_sh, gsems, ssems, nwave)

        plsc.subcore_barrier()

        @pl.when(c == 0)
        def _():
            pltpu.sync_copy(acc_sh.at[pl.ds(base, rows)], out0_hbm.at[pl.ds(base, rows)])

        @pl.when(c == 1)
        def _():
            pltpu.sync_copy(acc_sh.at[pl.ds(base, rows)], out1_hbm.at[pl.ds(base, rows)])

    return scat_kernel(src3d, dst3d, pa, pb)


def _sc_scatter(src3d, dst3d, p):
    ns, nch_s, k = src3d.shape
    nch = nch_s // _NC
    tpad, d = p.shape
    rows = tpad // _NS
    nwave = nch // (_G * _NB)

    @functools.partial(
        pl.kernel,
        out_type=(jax.ShapeDtypeStruct((tpad, d), jnp.float32),
                  jax.ShapeDtypeStruct((tpad, d), jnp.float32)),
        mesh=_mesh(),
        compiler_params=pltpu.CompilerParams(use_tc_tiling_on_sc=False),
        scratch_types=[
            pltpu.VMEM((nch, k), jnp.int32),
            pltpu.VMEM((nch, k), jnp.int32),
            pltpu.VMEM((_NB, k, d), jnp.float32),
            pltpu.VMEM((k, d), jnp.float32),
            pltpu.VMEM_SHARED((tpad, d), jnp.float32),
        ] + [pltpu.SemaphoreType.DMA] * (2 * _NB),
    )
    def scat_kernel(src_hbm, dst_hbm, p_hbm, out0_hbm, out1_hbm,
                    isrc2, idst2, rows_v, zt_v, acc_sh, *sems):
        gsems = sems[:_NB]
        ssems = sems[_NB:]
        c = lax.axis_index("c")
        s = lax.axis_index("s")
        w = s * _NC + c
        base = s * rows
        for r in range(k):
            for j in range(d // 16):
                zt_v[r, pl.ds(j * 16, 16)] = jnp.zeros((16,), jnp.float32)
        for i in range(rows // k):
            pltpu.sync_copy(zt_v, acc_sh.at[pl.ds(base + i * k, k)])
        pltpu.sync_copy(src_hbm.at[w // _NC, pl.ds((w % _NC) * nch, nch)], isrc2)
        pltpu.sync_copy(dst_hbm.at[w // _NC, pl.ds((w % _NC) * nch, nch)], idst2)
        plsc.subcore_barrier()
        _ring_scatter(p_hbm, isrc2, idst2, rows_v, acc_sh, gsems, ssems, nwave)
        plsc.subcore_barrier()

        @pl.when(c == 0)
        def _():
            pltpu.sync_copy(acc_sh.at[pl.ds(base, rows)], out0_hbm.at[pl.ds(base, rows)])

        @pl.when(c == 1)
        def _():
            pltpu.sync_copy(acc_sh.at[pl.ds(base, rows)], out1_hbm.at[pl.ds(base, rows)])

    return scat_kernel(src3d, dst3d, p)


def _tc_first(x, w1, dg0, dg1):
    n, d_in = x.shape
    d_out = w1.shape[1]
    dh = d_out // 2

    def body(x_ref, w_ref, d0_ref, d1_ref, pa_ref, pb_ref, dis_ref):
        deg = d0_ref[...] + d1_ref[...] + 1.0
        dis = lax.rsqrt(deg)
        h = jnp.dot(x_ref[...], w_ref[...], preferred_element_type=jnp.float32)
        p = h * dis
        pa_ref[...] = p[:, :dh]
        pb_ref[...] = p[:, dh:]
        dis_ref[...] = dis

    return pl.pallas_call(
        body,
        grid=(n // _R,),
        in_specs=[
            pl.BlockSpec((_R, d_in), lambda i: (i, 0)),
            pl.BlockSpec((d_in, d_out), lambda i: (0, 0)),
            pl.BlockSpec((_R, 1), lambda i: (i, 0)),
            pl.BlockSpec((_R, 1), lambda i: (i, 0)),
        ],
        out_specs=[
            pl.BlockSpec((_R, dh), lambda i: (i, 0)),
            pl.BlockSpec((_R, dh), lambda i: (i, 0)),
            pl.BlockSpec((_R, 1), lambda i: (i, 0)),
        ],
        out_shape=[
            jax.ShapeDtypeStruct((n, dh), jnp.float32),
            jax.ShapeDtypeStruct((n, dh), jnp.float32),
            jax.ShapeDtypeStruct((n, 1), jnp.float32),
        ],
    )(x, w1, dg0, dg1)


def _tc_mid(pa, pb, sa, sb, dis, ba, bb, wa, wb, split_out, out_width):
    n, dh = pa.shape
    d2 = wa.shape[1]

    def body(pa_ref, pb_ref, sa_ref, sb_ref, dis_ref, ba_ref, bb_ref,
             wa_ref, wb_ref, *out_refs):
        dis = dis_ref[...]
        ha = jnp.maximum((sa_ref[...] + pa_ref[...]) * dis + ba_ref[...], 0.0)
        hb = jnp.maximum((sb_ref[...] + pb_ref[...]) * dis + bb_ref[...], 0.0)
        q = (jnp.dot(ha, wa_ref[...], preferred_element_type=jnp.float32)
             + jnp.dot(hb, wb_ref[...], preferred_element_type=jnp.float32)) * dis
        if split_out:
            out_refs[0][...] = q[:, :d2 // 2]
            out_refs[1][...] = q[:, d2 // 2:]
        else:
            out_refs[0][...] = jnp.broadcast_to(q, (q.shape[0], out_width))

    if split_out:
        out_specs = [pl.BlockSpec((_R, d2 // 2), lambda i: (i, 0)),
                     pl.BlockSpec((_R, d2 // 2), lambda i: (i, 0))]
        out_shape = [jax.ShapeDtypeStruct((n, d2 // 2), jnp.float32),
                     jax.ShapeDtypeStruct((n, d2 // 2), jnp.float32)]
    else:
        out_specs = pl.BlockSpec((_R, out_width), lambda i: (i, 0))
        out_shape = jax.ShapeDtypeStruct((n, out_width), jnp.float32)

    return pl.pallas_call(
        body,
        grid=(n // _R,),
        in_specs=[
            pl.BlockSpec((_R, dh), lambda i: (i, 0)),
            pl.BlockSpec((_R, dh), lambda i: (i, 0)),
            pl.BlockSpec((_R, dh), lambda i: (i, 0)),
            pl.BlockSpec((_R, dh), lambda i: (i, 0)),
            pl.BlockSpec((_R, 1), lambda i: (i, 0)),
            pl.BlockSpec((1, dh), lambda i: (0, 0)),
            pl.BlockSpec((1, dh), lambda i: (0, 0)),
            pl.BlockSpec((dh, d2), lambda i: (0, 0)),
            pl.BlockSpec((dh, d2), lambda i: (0, 0)),
        ],
        out_specs=out_specs,
        out_shape=out_shape,
    )(pa, pb, sa, sb, dis, ba, bb, wa, wb)


def _tc_last(p16, a0, a1, dis, b3):
    n, d = p16.shape

    def body(p_ref, a0_ref, a1_ref, dis_ref, b_ref, o_ref):
        acc = a0_ref[...][:, :1] + a1_ref[...][:, :1] + p_ref[...][:, :1]
        v = acc * dis_ref[...] + b_ref[...]
        o_ref[...] = jax.nn.sigmoid(v)

    return pl.pallas_call(
        body,
        grid=(n // _R,),
        in_specs=[
            pl.BlockSpec((_R, d), lambda i: (i, 0)),
            pl.BlockSpec((_R, d), lambda i: (i, 0)),
            pl.BlockSpec((_R, d), lambda i: (i, 0)),
            pl.BlockSpec((_R, 1), lambda i: (i, 0)),
            pl.BlockSpec((1, 1), lambda i: (0, 0)),
        ],
        out_specs=pl.BlockSpec((_R, 1), lambda i: (i, 0)),
        out_shape=jax.ShapeDtypeStruct((n, 1), jnp.float32),
    )(p16, a0, a1, dis, b3)


def kernel(x, edge_index, W1, b1, W2, b2, W3, b3):
    n, d_feat = x.shape
    e = edge_index.shape[1]
    tpad = ((n + _R - 1) // _R) * _R
    assert tpad % (_NS * 8) == 0

    ew = e // _NW
    ew_pad = ((ew + _K * _NB - 1) // (_K * _NB)) * (_K * _NB)
    epad = ew_pad * _NW
    nch_w = ew_pad // _K
    nch_s = (epad // _NS) // _K
    n_dummy = ew_pad - ew
    assert e % _NW == 0 and n_dummy <= tpad - n
    assert nch_s % (_G * _NB) == 0 and nch_w % (_G * _NB) == 0
    pad_rows = n + jnp.arange(n_dummy, dtype=jnp.int32) % (tpad - n)

    def _prep(idx):
        w2 = idx.astype(jnp.int32).reshape(_NW, ew)
        padb = jnp.broadcast_to(pad_rows, (_NW, n_dummy))
        return jnp.concatenate([w2, padb], axis=1).reshape(_NS, nch_s, _K)

    src3d = _prep(edge_index[0])
    dst3d = _prep(edge_index[1])
    x_pad = jnp.zeros((tpad, d_feat), jnp.float32).at[:n].set(x)

    d1h = W1.shape[1] // 2
    d2h = W2.shape[1] // 2

    degp0, degp1 = _sc_degree(dst3d, tpad)
    dg0 = degp0.reshape(tpad, 1)
    dg1 = degp1.reshape(tpad, 1)
    p1a, p1b, dis = _tc_first(x_pad, W1, dg0, dg1)
    s1a, s1b = _sc_scatter_split(src3d, dst3d, p1a, p1b)
    p2a, p2b = _tc_mid(p1a, p1b, s1a, s1b, dis,
                       b1.reshape(1, -1)[:, :d1h], b1.reshape(1, -1)[:, d1h:],
                       W2[:d1h], W2[d1h:], True, 0)
    s2a, s2b = _sc_scatter_split(src3d, dst3d, p2a, p2b)
    p3 = _tc_mid(p2a, p2b, s2a, s2b, dis,
                 b2.reshape(1, -1)[:, :d2h], b2.reshape(1, -1)[:, d2h:],
                 W3[:d2h], W3[d2h:], False, 16)
    a0, a1 = _sc_scatter(src3d, dst3d, p3)
    out = _tc_last(p3, a0, a1, dis, b3.reshape(1, 1))
    return out[:n]


# speedup vs baseline: 1.0569x; 1.0235x over previous
import functools

import jax
import jax.numpy as jnp
from jax import lax
from jax.experimental import pallas as pl
from jax.experimental.pallas import tpu as pltpu
from jax.experimental.pallas import tpu_sc as plsc

_NC, _NS = 2, 16
_NW = _NC * _NS
_K = 128
_G = 1
_NB = 8
_R = 2048


def _mesh():
    return plsc.VectorSubcoreMesh(core_axis_name="c", subcore_axis_name="s",
                                  num_cores=_NC, num_subcores=_NS)


def _sc_degree(dst3d, tpad):
    ns, nch_s, k = dst3d.shape
    nch = nch_s // _NC
    rows = tpad // _NS
    nwave = nch // (_G * _NB)

    @functools.partial(
        pl.kernel,
        out_type=(jax.ShapeDtypeStruct((tpad,), jnp.float32),
                  jax.ShapeDtypeStruct((tpad,), jnp.float32)),
        mesh=_mesh(),
        compiler_params=pltpu.CompilerParams(use_tc_tiling_on_sc=False),
        scratch_types=[
            pltpu.VMEM((nch, k), jnp.int32),
            pltpu.VMEM((k,), jnp.float32),
            pltpu.VMEM((k,), jnp.float32),
            pltpu.VMEM_SHARED((tpad,), jnp.float32),
            pltpu.SemaphoreType.DMA,
        ],
    )
    def deg_kernel(dst_hbm, out0_hbm, out1_hbm, idst2, ones_v, zv, acc_sh, ssem):
        c = lax.axis_index("c")
        s = lax.axis_index("s")
        w = s * _NC + c
        base = s * rows
        for i in range(k // 16):
            ones_v[pl.ds(i * 16, 16)] = jnp.full((16,), 1.0, jnp.float32)
            zv[pl.ds(i * 16, 16)] = jnp.zeros((16,), jnp.float32)
        for i in range(rows // k):
            pltpu.sync_copy(zv, acc_sh.at[pl.ds(base + i * k, k)])
        pltpu.sync_copy(dst_hbm.at[w // _NC, pl.ds((w % _NC) * nch, nch)], idst2)
        plsc.subcore_barrier()

        def wave(t, carry):
            g0 = t * _NB
            descs = [
                pltpu.async_copy(ones_v, acc_sh.at[idst2.at[g0 + b]], ssem, add=True)
                for b in range(_NB)
            ]
            for dsc in descs:
                dsc.wait()
            return carry

        lax.fori_loop(0, nwave, wave, 0)
        plsc.subcore_barrier()

        @pl.when(c == 0)
        def _():
            pltpu.sync_copy(acc_sh.at[pl.ds(base, rows)], out0_hbm.at[pl.ds(base, rows)])

        @pl.when(c == 1)
        def _():
            pltpu.sync_copy(acc_sh.at[pl.ds(base, rows)], out1_hbm.at[pl.ds(base, rows)])

    return deg_kernel(dst3d)


def _ring_scatter(p_hbm, isrc2, idst2, rows_v, acc_sh, gsems, ssems, nwave):
    for b in range(_NB):
        pltpu.async_copy(p_hbm.at[isrc2.at[b]], rows_v.at[b], gsems[b])

    def wave(t, carry):
        g0 = t * _NB
        sds = []
        for b in range(_NB):
            pltpu.make_async_copy(
                p_hbm.at[isrc2.at[g0 + b]], rows_v.at[b], gsems[b]).wait()
            sds.append(pltpu.async_copy(
                rows_v.at[b], acc_sh.at[idst2.at[g0 + b]], ssems[b], add=True))
        for b in range(_NB):
            sds[b].wait()

            @pl.when(t < nwave - 1)
            def _(b=b, g0=g0):
                pltpu.async_copy(
                    p_hbm.at[isrc2.at[g0 + _NB + b]], rows_v.at[b], gsems[b])
        return carry

    lax.fori_loop(0, nwave, wave, 0)


def _sc_scatter_split(src3d, dst3d, pa, pb):
    ns, nch, k = src3d.shape
    tpad, dh = pa.shape
    rows = tpad // _NS
    nwave = nch // (_G * _NB)

    @functools.partial(
        pl.kernel,
        out_type=(jax.ShapeDtypeStruct((tpad, dh), jnp.float32),
                  jax.ShapeDtypeStruct((tpad, dh), jnp.float32)),
        mesh=_mesh(),
        compiler_params=pltpu.CompilerParams(use_tc_tiling_on_sc=False),
        scratch_types=[
            pltpu.VMEM((nch, k), jnp.int32),
            pltpu.VMEM((nch, k), jnp.int32),
            pltpu.VMEM((_NB, k, dh), jnp.float32),
            pltpu.VMEM((k, dh), jnp.float32),
            pltpu.VMEM_SHARED((tpad, dh), jnp.float32),
        ] + [pltpu.SemaphoreType.DMA] * (2 * _NB),
    )
    def scat_kernel(src_hbm, dst_hbm, pa_hbm, pb_hbm, out0_hbm, out1_hbm,
                    isrc2, idst2, rows_v, zt_v, acc_sh, *sems):
        gsems = sems[:_NB]
        ssems = sems[_NB:]
        c = lax.axis_index("c")
        s = lax.axis_index("s")
        base = s * rows
        for r in range(k):
            for j in range(dh // 16):
                zt_v[r, pl.ds(j * 16, 16)] = jnp.zeros((16,), jnp.float32)
        for i in range(rows // k):
            pltpu.sync_copy(zt_v, acc_sh.at[pl.ds(base + i * k, k)])
        pltpu.sync_copy(src_hbm.at[s], isrc2)
        pltpu.sync_copy(dst_hbm.at[s], idst2)
        plsc.subcore_barrier()

        @pl.when(c == 0)
        def _():
            _ring_scatter(pa_hbm, isrc2, idst2, rows_v, acc_sh, gsems, ssems, nwave)

        @pl.when(c == 1)
        def _():
            _ring_scatter(pb_hbm, isrc2, idst2, rows_v, acc_sh, gsems, ssems, nwave)

        plsc.subcore_barrier()

        @pl.when(c == 0)
        def _():
            pltpu.sync_copy(acc_sh.at[pl.ds(base, rows)], out0_hbm.at[pl.ds(base, rows)])

        @pl.when(c == 1)
        def _():
            pltpu.sync_copy(acc_sh.at[pl.ds(base, rows)], out1_hbm.at[pl.ds(base, rows)])

    return scat_kernel(src3d, dst3d, pa, pb)


def _sc_scatter(src3d, dst3d, p):
    ns, nch_s, k = src3d.shape
    nch = nch_s // _NC
    tpad, d = p.shape
    rows = tpad // _NS
    nwave = nch // (_G * _NB)

    @functools.partial(
        pl.kernel,
        out_type=(jax.ShapeDtypeStruct((tpad, d), jnp.float32),
                  jax.ShapeDtypeStruct((tpad, d), jnp.float32)),
        mesh=_mesh(),
        compiler_params=pltpu.CompilerParams(use_tc_tiling_on_sc=False),
        scratch_types=[
            pltpu.VMEM((nch, k), jnp.int32),
            pltpu.VMEM((nch, k), jnp.int32),
            pltpu.VMEM((_NB, k, d), jnp.float32),
            pltpu.VMEM((k, d), jnp.float32),
            pltpu.VMEM_SHARED((tpad, d), jnp.float32),
        ] + [pltpu.SemaphoreType.DMA] * (2 * _NB),
    )
    def scat_kernel(src_hbm, dst_hbm, p_hbm, out0_hbm, out1_hbm,
                    isrc2, idst2, rows_v, zt_v, acc_sh, *sems):
        gsems = sems[:_NB]
        ssems = sems[_NB:]
        c = lax.axis_index("c")
        s = lax.axis_index("s")
        w = s * _NC + c
        base = s * rows
        for r in range(k):
            for j in range(d // 16):
                zt_v[r, pl.ds(j * 16, 16)] = jnp.zeros((16,), jnp.float32)
        for i in range(rows // k):
            pltpu.sync_copy(zt_v, acc_sh.at[pl.ds(base + i * k, k)])
        pltpu.sync_copy(src_hbm.at[w // _NC, pl.ds((w % _NC) * nch, nch)], isrc2)
        pltpu.sync_copy(dst_hbm.at[w // _NC, pl.ds((w % _NC) * nch, nch)], idst2)
        plsc.subcore_barrier()
        _ring_scatter(p_hbm, isrc2, idst2, rows_v, acc_sh, gsems, ssems, nwave)
        plsc.subcore_barrier()

        @pl.when(c == 0)
        def _():
            pltpu.sync_copy(acc_sh.at[pl.ds(base, rows)], out0_hbm.at[pl.ds(base, rows)])

        @pl.when(c == 1)
        def _():
            pltpu.sync_copy(acc_sh.at[pl.ds(base, rows)], out1_hbm.at[pl.ds(base, rows)])

    return scat_kernel(src3d, dst3d, p)


def _tc_first(x, w1, dg0, dg1):
    n, d_in = x.shape
    d_out = w1.shape[1]
    dh = d_out // 2

    def body(x_ref, w_ref, d0_ref, d1_ref, pa_ref, pb_ref, dis_ref):
        deg = d0_ref[...] + d1_ref[...] + 1.0
        dis = lax.rsqrt(deg)
        h = jnp.dot(x_ref[...], w_ref[...], preferred_element_type=jnp.float32)
        p = h * dis
        pa_ref[...] = p[:, :dh]
        pb_ref[...] = p[:, dh:]
        dis_ref[...] = dis

    return pl.pallas_call(
        body,
        grid=(n // _R,),
        in_specs=[
            pl.BlockSpec((_R, d_in), lambda i: (i, 0)),
            pl.BlockSpec((d_in, d_out), lambda i: (0, 0)),
            pl.BlockSpec((_R, 1), lambda i: (i, 0)),
            pl.BlockSpec((_R, 1), lambda i: (i, 0)),
        ],
        out_specs=[
            pl.BlockSpec((_R, dh), lambda i: (i, 0)),
            pl.BlockSpec((_R, dh), lambda i: (i, 0)),
            pl.BlockSpec((_R, 1), lambda i: (i, 0)),
        ],
        out_shape=[
            jax.ShapeDtypeStruct((n, dh), jnp.float32),
            jax.ShapeDtypeStruct((n, dh), jnp.float32),
            jax.ShapeDtypeStruct((n, 1), jnp.float32),
        ],
    )(x, w1, dg0, dg1)


def _tc_mid(pa, pb, sa, sb, dis, ba, bb, wa, wb, split_out, out_width):
    n, dh = pa.shape
    d2 = wa.shape[1]

    def body(pa_ref, pb_ref, sa_ref, sb_ref, dis_ref, ba_ref, bb_ref,
             wa_ref, wb_ref, *out_refs):
        dis = dis_ref[...]
        ha = jnp.maximum((sa_ref[...] + pa_ref[...]) * dis + ba_ref[...], 0.0)
        hb = jnp.maximum((sb_ref[...] + pb_ref[...]) * dis + bb_ref[...], 0.0)
        q = (jnp.dot(ha, wa_ref[...], preferred_element_type=jnp.float32)
             + jnp.dot(hb, wb_ref[...], preferred_element_type=jnp.float32)) * dis
        if split_out:
            out_refs[0][...] = q[:, :d2 // 2]
            out_refs[1][...] = q[:, d2 // 2:]
        else:
            out_refs[0][...] = jnp.broadcast_to(q, (q.shape[0], out_width))

    if split_out:
        out_specs = [pl.BlockSpec((_R, d2 // 2), lambda i: (i, 0)),
                     pl.BlockSpec((_R, d2 // 2), lambda i: (i, 0))]
        out_shape = [jax.ShapeDtypeStruct((n, d2 // 2), jnp.float32),
                     jax.ShapeDtypeStruct((n, d2 // 2), jnp.float32)]
    else:
        out_specs = pl.BlockSpec((_R, out_width), lambda i: (i, 0))
        out_shape = jax.ShapeDtypeStruct((n, out_width), jnp.float32)

    return pl.pallas_call(
        body,
        grid=(n // _R,),
        in_specs=[
            pl.BlockSpec((_R, dh), lambda i: (i, 0)),
            pl.BlockSpec((_R, dh), lambda i: (i, 0)),
            pl.BlockSpec((_R, dh), lambda i: (i, 0)),
            pl.BlockSpec((_R, dh), lambda i: (i, 0)),
            pl.BlockSpec((_R, 1), lambda i: (i, 0)),
            pl.BlockSpec((1, dh), lambda i: (0, 0)),
            pl.BlockSpec((1, dh), lambda i: (0, 0)),
            pl.BlockSpec((dh, d2), lambda i: (0, 0)),
            pl.BlockSpec((dh, d2), lambda i: (0, 0)),
        ],
        out_specs=out_specs,
        out_shape=out_shape,
    )(pa, pb, sa, sb, dis, ba, bb, wa, wb)


def _tc_last(p16, a0, a1, dis, b3):
    n, d = p16.shape

    def body(p_ref, a0_ref, a1_ref, dis_ref, b_ref, o_ref):
        acc = a0_ref[...][:, :1] + a1_ref[...][:, :1] + p_ref[...][:, :1]
        v = acc * dis_ref[...] + b_ref[...]
        o_ref[...] = jax.nn.sigmoid(v)

    return pl.pallas_call(
        body,
        grid=(n // _R,),
        in_specs=[
            pl.BlockSpec((_R, d), lambda i: (i, 0)),
            pl.BlockSpec((_R, d), lambda i: (i, 0)),
            pl.BlockSpec((_R, d), lambda i: (i, 0)),
            pl.BlockSpec((_R, 1), lambda i: (i, 0)),
            pl.BlockSpec((1, 1), lambda i: (0, 0)),
        ],
        out_specs=pl.BlockSpec((_R, 1), lambda i: (i, 0)),
        out_shape=jax.ShapeDtypeStruct((n, 1), jnp.float32),
    )(p16, a0, a1, dis, b3)


def kernel(x, edge_index, W1, b1, W2, b2, W3, b3):
    n, d_feat = x.shape
    e = edge_index.shape[1]
    tpad = ((n + _R - 1) // _R) * _R
    assert tpad % (_NS * 8) == 0

    ew = e // _NW
    ew_pad = ((ew + _K * _NB - 1) // (_K * _NB)) * (_K * _NB)
    epad = ew_pad * _NW
    nch_w = ew_pad // _K
    nch_s = (epad // _NS) // _K
    n_dummy = ew_pad - ew
    assert e % _NW == 0 and n_dummy <= tpad - n
    assert nch_s % (_G * _NB) == 0 and nch_w % (_G * _NB) == 0
    pad_rows = n + jnp.arange(n_dummy, dtype=jnp.int32) % (tpad - n)

    def _prep(idx):
        w2 = idx.astype(jnp.int32).reshape(_NW, ew)
        padb = jnp.broadcast_to(pad_rows, (_NW, n_dummy))
        return jnp.concatenate([w2, padb], axis=1).reshape(_NS, nch_s, _K)

    src3d = _prep(edge_index[0])
    dst3d = _prep(edge_index[1])
    x_pad = jnp.zeros((tpad, d_feat), jnp.float32).at[:n].set(x)

    d1h = W1.shape[1] // 2
    d2h = W2.shape[1] // 2

    degp0, degp1 = _sc_degree(dst3d, tpad)
    dg0 = degp0.reshape(tpad, 1)
    dg1 = degp1.reshape(tpad, 1)
    p1a, p1b, dis = _tc_first(x_pad, W1, dg0, dg1)
    s1a, s1b = _sc_scatter_split(src3d, dst3d, p1a, p1b)
    p2a, p2b = _tc_mid(p1a, p1b, s1a, s1b, dis,
                       b1.reshape(1, -1)[:, :d1h], b1.reshape(1, -1)[:, d1h:],
                       W2[:d1h], W2[d1h:], True, 0)
    s2a, s2b = _sc_scatter_split(src3d, dst3d, p2a, p2b)
    p3 = _tc_mid(p2a, p2b, s2a, s2b, dis,
                 b2.reshape(1, -1)[:, :d2h], b2.reshape(1, -1)[:, d2h:],
                 W3[:d2h], W3[d2h:], False, 16)
    a0, a1 = _sc_scatter(src3d, dst3d, p3)
    out = _tc_last(p3, a0, a1, dis, b3.reshape(1, 1))
    return out[:n]
